# Initial kernel scaffold; baseline (speedup 1.0000x reference)
import jax
import jax.numpy as jnp
from jax.experimental import pallas as pl


def kernel(x, codebooks):
    raise NotImplementedError("write your pallas kernel here")


# speedup vs baseline: 1.2965x; 1.2965x over previous
import functools

import jax
import jax.numpy as jnp
from jax.experimental import pallas as pl
from jax.experimental.pallas import tpu as pltpu

NUM_Q = 4
K = 512
D = 32
ROWS = 64 * 1024
BLOCK = 2048


def _rvq_kernel(x_ref, cbt_ref, cb_ref, qout_ref, idx_ref, loss_ref):
    step = pl.program_id(0)
    nsteps = pl.num_programs(0)

    @pl.when(step == 0)
    def _init():
        loss_ref[...] = jnp.zeros_like(loss_ref)

    x0 = x_ref[...]
    r = x0
    qacc = jnp.zeros_like(x0)
    iota_k = jax.lax.broadcasted_iota(jnp.int32, (BLOCK, K), 1)
    q_iota = jax.lax.broadcasted_iota(jnp.int32, (1, NUM_Q), 1)

    for q in range(NUM_Q):
        cbt = cbt_ref[q]
        cn = jnp.sum(cbt * cbt, axis=0, keepdims=True)
        rn = jnp.sum(r * r, axis=1, keepdims=True)
        t = jax.lax.dot_general(
            r.astype(jnp.bfloat16), cbt.astype(jnp.bfloat16),
            (((1,), (0,)), ((), ())),
            preferred_element_type=jnp.float32)
        dist = rn - 2.0 * t + cn
        m = jnp.min(dist, axis=1, keepdims=True)
        idx = jnp.min(jnp.where(dist == m, iota_k, K), axis=1,
                      keepdims=True)
        onehot = (iota_k == idx).astype(jnp.float32)
        q_raw = jax.lax.dot_general(
            onehot, cb_ref[q], (((1,), (0,)), ((), ())),
            precision=jax.lax.Precision.HIGHEST,
            preferred_element_type=jnp.float32)
        quant = r + (q_raw - r)
        s = jnp.sum((q_raw - r) * (q_raw - r))
        r = r - quant
        qacc = qacc + quant
        idx_ref[:, q:q + 1] = idx
        loss_ref[...] += jnp.where(q_iota == q, s, 0.0)

    qout_ref[...] = qacc

    @pl.when(step == nsteps - 1)
    def _scale():
        loss_ref[...] = loss_ref[...] * (1.25 / float(ROWS * D))


@functools.partial(jax.jit, static_argnames=("interpret",))
def kernel(x, codebooks, interpret=False):
    b, n, d = x.shape
    xf = x.reshape(-1, d)
    cbt = jnp.transpose(codebooks, (0, 2, 1))
    grid = (ROWS // BLOCK,)
    qout, idx, loss = pl.pallas_call(
        _rvq_kernel,
        grid=grid,
        in_specs=[
            pl.BlockSpec((BLOCK, D), lambda i: (i, 0)),
            pl.BlockSpec((NUM_Q, D, K), lambda i: (0, 0, 0)),
            pl.BlockSpec((NUM_Q, K, D), lambda i: (0, 0, 0)),
        ],
        out_specs=[
            pl.BlockSpec((BLOCK, D), lambda i: (i, 0)),
            pl.BlockSpec((BLOCK, NUM_Q), lambda i: (i, 0)),
            pl.BlockSpec((1, NUM_Q), lambda i: (0, 0)),
        ],
        out_shape=[
            jax.ShapeDtypeStruct((ROWS, D), jnp.float32),
            jax.ShapeDtypeStruct((ROWS, NUM_Q), jnp.int32),
            jax.ShapeDtypeStruct((1, NUM_Q), jnp.float32),
        ],
        compiler_params=pltpu.CompilerParams(
            dimension_semantics=("arbitrary",),
        ),
        interpret=interpret,
    )(xf, cbt, codebooks)
    quantized_out = qout.reshape(b, n, d)
    all_indices = idx.reshape(b, n, NUM_Q)
    all_num_expired = jnp.zeros((NUM_Q,), dtype=jnp.int32)
    all_losses = loss.reshape(NUM_Q)
    return (quantized_out, all_indices, all_num_expired, all_losses)


# speedup vs baseline: 2.1665x; 1.6710x over previous
import functools

import jax
import jax.numpy as jnp
from jax.experimental import pallas as pl
from jax.experimental.pallas import tpu as pltpu

NUM_Q = 4
K = 512
D = 32
ROWS = 64 * 1024
BLOCK = 2048


def _rvq_kernel(x_ref, cbt_ref, cb_ref, qout_ref, idx_ref, loss_ref):
    step = pl.program_id(0)
    nsteps = pl.num_programs(0)

    @pl.when(step == 0)
    def _init():
        loss_ref[...] = jnp.zeros_like(loss_ref)

    x0 = x_ref[...]
    r = x0
    qacc = jnp.zeros_like(x0)
    iota_k = jax.lax.broadcasted_iota(jnp.int32, (BLOCK, K), 1)
    q_iota = jax.lax.broadcasted_iota(jnp.int32, (1, NUM_Q), 1)

    for q in range(NUM_Q):
        cbt = cbt_ref[q]
        cn = jnp.sum(cbt * cbt, axis=0, keepdims=True)
        rn = jnp.sum(r * r, axis=1, keepdims=True)
        t = jax.lax.dot_general(
            r.astype(jnp.bfloat16), cbt.astype(jnp.bfloat16),
            (((1,), (0,)), ((), ())),
            preferred_element_type=jnp.float32)
        dist = rn - 2.0 * t + cn
        m = jnp.min(dist, axis=1, keepdims=True)
        idx = jnp.min(jnp.where(dist == m, iota_k, K), axis=1,
                      keepdims=True)
        onehot = (iota_k == idx).astype(jnp.bfloat16)
        cb = cb_ref[q]
        c_hi = cb.astype(jnp.bfloat16)
        rest = cb - c_hi.astype(jnp.float32)
        c_mid = rest.astype(jnp.bfloat16)
        c_lo = (rest - c_mid.astype(jnp.float32)).astype(jnp.bfloat16)

        def _gmm(c_part):
            return jax.lax.dot_general(
                onehot, c_part, (((1,), (0,)), ((), ())),
                preferred_element_type=jnp.float32)

        q_raw = _gmm(c_hi) + (_gmm(c_mid) + _gmm(c_lo))
        quant = r + (q_raw - r)
        s = jnp.sum((q_raw - r) * (q_raw - r))
        r = r - quant
        qacc = qacc + quant
        idx_ref[:, q:q + 1] = idx
        loss_ref[...] += jnp.where(q_iota == q, s, 0.0)

    qout_ref[...] = qacc

    @pl.when(step == nsteps - 1)
    def _scale():
        loss_ref[...] = loss_ref[...] * (1.25 / float(ROWS * D))


@functools.partial(jax.jit, static_argnames=("interpret",))
def kernel(x, codebooks, interpret=False):
    b, n, d = x.shape
    xf = x.reshape(-1, d)
    cbt = jnp.transpose(codebooks, (0, 2, 1))
    grid = (ROWS // BLOCK,)
    qout, idx, loss = pl.pallas_call(
        _rvq_kernel,
        grid=grid,
        in_specs=[
            pl.BlockSpec((BLOCK, D), lambda i: (i, 0)),
            pl.BlockSpec((NUM_Q, D, K), lambda i: (0, 0, 0)),
            pl.BlockSpec((NUM_Q, K, D), lambda i: (0, 0, 0)),
        ],
        out_specs=[
            pl.BlockSpec((BLOCK, D), lambda i: (i, 0)),
            pl.BlockSpec((BLOCK, NUM_Q), lambda i: (i, 0)),
            pl.BlockSpec((1, NUM_Q), lambda i: (0, 0)),
        ],
        out_shape=[
            jax.ShapeDtypeStruct((ROWS, D), jnp.float32),
            jax.ShapeDtypeStruct((ROWS, NUM_Q), jnp.int32),
            jax.ShapeDtypeStruct((1, NUM_Q), jnp.float32),
        ],
        compiler_params=pltpu.CompilerParams(
            dimension_semantics=("arbitrary",),
        ),
        interpret=interpret,
    )(xf, cbt, codebooks)
    quantized_out = qout.reshape(b, n, d)
    all_indices = idx.reshape(b, n, NUM_Q)
    all_num_expired = jnp.zeros((NUM_Q,), dtype=jnp.int32)
    all_losses = loss.reshape(NUM_Q)
    return (quantized_out, all_indices, all_num_expired, all_losses)


# speedup vs baseline: 3.1970x; 1.4757x over previous
import functools

import jax
import jax.numpy as jnp
from jax.experimental import pallas as pl
from jax.experimental.pallas import tpu as pltpu

NUM_Q = 4
K = 512
D = 32
ROWS = 64 * 1024
BLOCK = 4096


def _rvq_kernel(x_ref, cbt_ref, cb_ref, qout_ref, idx_ref, loss_ref):
    step = pl.program_id(0)
    nsteps = pl.num_programs(0)

    @pl.when(step == 0)
    def _init():
        loss_ref[...] = jnp.zeros_like(loss_ref)

    x0 = x_ref[...]
    r = x0
    qacc = jnp.zeros_like(x0)
    iota_k = jax.lax.broadcasted_iota(jnp.int32, (BLOCK, K), 1)
    q_iota = jax.lax.broadcasted_iota(jnp.int32, (1, NUM_Q), 1)

    for q in range(NUM_Q):
        cbt = cbt_ref[q]
        cn = jnp.sum(cbt * cbt, axis=0, keepdims=True)
        rn = jnp.sum(r * r, axis=1, keepdims=True)
        t = jax.lax.dot_general(
            r.astype(jnp.bfloat16), cbt.astype(jnp.bfloat16),
            (((1,), (0,)), ((), ())),
            preferred_element_type=jnp.float32)
        dist = rn - 2.0 * t + cn
        m = jnp.min(dist, axis=1, keepdims=True)
        idx = jnp.min(jnp.where(dist == m, iota_k, K), axis=1,
                      keepdims=True)
        onehot = (iota_k == idx).astype(jnp.bfloat16)
        cb = cb_ref[q]
        c_hi = cb.astype(jnp.bfloat16)
        rest = cb - c_hi.astype(jnp.float32)
        c_mid = rest.astype(jnp.bfloat16)
        c_lo = (rest - c_mid.astype(jnp.float32)).astype(jnp.bfloat16)
        c_cat = jnp.concatenate([c_hi, c_mid, c_lo], axis=1)
        p = jax.lax.dot_general(
            onehot, c_cat, (((1,), (0,)), ((), ())),
            preferred_element_type=jnp.float32)
        q_raw = p[:, :D] + (p[:, D:2 * D] + p[:, 2 * D:])
        quant = r + (q_raw - r)
        s = jnp.sum((q_raw - r) * (q_raw - r))
        r = r - quant
        qacc = qacc + quant
        idx_ref[:, q:q + 1] = idx
        loss_ref[...] += jnp.where(q_iota == q, s, 0.0)

    qout_ref[...] = qacc

    @pl.when(step == nsteps - 1)
    def _scale():
        loss_ref[...] = loss_ref[...] * (1.25 / float(ROWS * D))


@functools.partial(jax.jit, static_argnames=("interpret",))
def kernel(x, codebooks, interpret=False):
    b, n, d = x.shape
    xf = x.reshape(-1, d)
    cbt = jnp.transpose(codebooks, (0, 2, 1))
    grid = (ROWS // BLOCK,)
    qout, idx, loss = pl.pallas_call(
        _rvq_kernel,
        grid=grid,
        in_specs=[
            pl.BlockSpec((BLOCK, D), lambda i: (i, 0)),
            pl.BlockSpec((NUM_Q, D, K), lambda i: (0, 0, 0)),
            pl.BlockSpec((NUM_Q, K, D), lambda i: (0, 0, 0)),
        ],
        out_specs=[
            pl.BlockSpec((BLOCK, D), lambda i: (i, 0)),
            pl.BlockSpec((BLOCK, NUM_Q), lambda i: (i, 0)),
            pl.BlockSpec((1, NUM_Q), lambda i: (0, 0)),
        ],
        out_shape=[
            jax.ShapeDtypeStruct((ROWS, D), jnp.float32),
            jax.ShapeDtypeStruct((ROWS, NUM_Q), jnp.int32),
            jax.ShapeDtypeStruct((1, NUM_Q), jnp.float32),
        ],
        compiler_params=pltpu.CompilerParams(
            dimension_semantics=("arbitrary",),
        ),
        interpret=interpret,
    )(xf, cbt, codebooks)
    quantized_out = qout.reshape(b, n, d)
    all_indices = idx.reshape(b, n, NUM_Q)
    all_num_expired = jnp.zeros((NUM_Q,), dtype=jnp.int32)
    all_losses = loss.reshape(NUM_Q)
    return (quantized_out, all_indices, all_num_expired, all_losses)


# speedup vs baseline: 4.0798x; 1.2761x over previous
import functools

import jax
import jax.numpy as jnp
from jax.experimental import pallas as pl
from jax.experimental.pallas import tpu as pltpu

NUM_Q = 4
K = 512
D = 32
ROWS = 64 * 1024
BLOCK = 4096


def _rvq_kernel(x_ref, cbt_ref, cb_ref, qout_ref, idx_ref, loss_ref):
    step = pl.program_id(0)
    nsteps = pl.num_programs(0)

    @pl.when(step == 0)
    def _init():
        loss_ref[...] = jnp.zeros_like(loss_ref)

    x0 = x_ref[...]
    r = x0
    qacc = jnp.zeros_like(x0)
    iota_k = jax.lax.broadcasted_iota(jnp.int32, (BLOCK, K), 1)
    q_iota = jax.lax.broadcasted_iota(jnp.int32, (1, NUM_Q), 1)

    ones3 = jnp.ones((BLOCK, 3), dtype=jnp.bfloat16)

    for q in range(NUM_Q):
        cbt = cbt_ref[q]
        cn = jnp.sum(cbt * cbt, axis=0, keepdims=True)
        cn_hi = cn.astype(jnp.bfloat16)
        cn_rest = cn - cn_hi.astype(jnp.float32)
        cn_mid = cn_rest.astype(jnp.bfloat16)
        cn_lo = (cn_rest - cn_mid.astype(jnp.float32)).astype(jnp.bfloat16)
        b_aug = jnp.concatenate(
            [cbt.astype(jnp.bfloat16) * jnp.bfloat16(-2.0),
             cn_hi, cn_mid, cn_lo], axis=0)
        a_aug = jnp.concatenate(
            [r.astype(jnp.bfloat16), ones3], axis=1)
        dist = jax.lax.dot_general(
            a_aug, b_aug, (((1,), (0,)), ((), ())),
            preferred_element_type=jnp.float32)
        m = jnp.min(dist, axis=1, keepdims=True)
        idx = jnp.min(jnp.where(dist == m, iota_k, K), axis=1,
                      keepdims=True)
        onehot = (iota_k == idx).astype(jnp.bfloat16)
        cb = cb_ref[q]
        c_hi = cb.astype(jnp.bfloat16)
        rest = cb - c_hi.astype(jnp.float32)
        c_mid = rest.astype(jnp.bfloat16)
        c_lo = (rest - c_mid.astype(jnp.float32)).astype(jnp.bfloat16)
        c_cat = jnp.concatenate([c_hi, c_mid, c_lo], axis=1)
        p = jax.lax.dot_general(
            onehot, c_cat, (((1,), (0,)), ((), ())),
            preferred_element_type=jnp.float32)
        q_raw = p[:, :D] + (p[:, D:2 * D] + p[:, 2 * D:])
        quant = r + (q_raw - r)
        s = jnp.sum((q_raw - r) * (q_raw - r))
        r = r - quant
        qacc = qacc + quant
        idx_ref[:, q:q + 1] = idx
        loss_ref[...] += jnp.where(q_iota == q, s, 0.0)

    qout_ref[...] = qacc

    @pl.when(step == nsteps - 1)
    def _scale():
        loss_ref[...] = loss_ref[...] * (1.25 / float(ROWS * D))


@functools.partial(jax.jit, static_argnames=("interpret",))
def kernel(x, codebooks, interpret=False):
    b, n, d = x.shape
    xf = x.reshape(-1, d)
    cbt = jnp.transpose(codebooks, (0, 2, 1))
    grid = (ROWS // BLOCK,)
    qout, idx, loss = pl.pallas_call(
        _rvq_kernel,
        grid=grid,
        in_specs=[
            pl.BlockSpec((BLOCK, D), lambda i: (i, 0)),
            pl.BlockSpec((NUM_Q, D, K), lambda i: (0, 0, 0)),
            pl.BlockSpec((NUM_Q, K, D), lambda i: (0, 0, 0)),
        ],
        out_specs=[
            pl.BlockSpec((BLOCK, D), lambda i: (i, 0)),
            pl.BlockSpec((BLOCK, NUM_Q), lambda i: (i, 0)),
            pl.BlockSpec((1, NUM_Q), lambda i: (0, 0)),
        ],
        out_shape=[
            jax.ShapeDtypeStruct((ROWS, D), jnp.float32),
            jax.ShapeDtypeStruct((ROWS, NUM_Q), jnp.int32),
            jax.ShapeDtypeStruct((1, NUM_Q), jnp.float32),
        ],
        compiler_params=pltpu.CompilerParams(
            dimension_semantics=("arbitrary",),
        ),
        interpret=interpret,
    )(xf, cbt, codebooks)
    quantized_out = qout.reshape(b, n, d)
    all_indices = idx.reshape(b, n, NUM_Q)
    all_num_expired = jnp.zeros((NUM_Q,), dtype=jnp.int32)
    all_losses = loss.reshape(NUM_Q)
    return (quantized_out, all_indices, all_num_expired, all_losses)
